# baseline (speedup 1.0000x reference)
import functools

import jax
import jax.numpy as jnp
from jax.experimental import pallas as pl
from jax.experimental.pallas import tpu as pltpu


def _fused_kernel(x_ref, w_ref, b_ref, g_ref, be_ref, mm_ref, mp_ref, o_ref,
                  xe_ref, xm_ref, xp_ref, p_ref, y_ref, s_ref, q_ref, *,
                  nb1, nb2, c1, cin, kh, kw, h, w, m, eps):
    step = pl.program_id(0)
    hw = h * w
    border = 128

    @pl.when(step < c1)
    def conv_phase():
        @pl.when(step == 0)
        def _init():
            xe_ref[...] = jnp.zeros_like(xe_ref)
            xm_ref[...] = jnp.zeros_like(xm_ref)
            xp_ref[...] = jnp.zeros_like(xp_ref)
            s_ref[...] = jnp.zeros_like(s_ref)
            q_ref[...] = jnp.zeros_like(q_ref)

        hw0 = min((hw // 2 + 127) // 128 * 128, hw)
        halves = ((0, hw0),) if hw0 == hw else ((0, hw0), (hw0, hw - hw0))
        for b in range(nb1):
            xb = x_ref[b].astype(jnp.bfloat16)
            xe_ref[:, border:border + hw] = xb
            xm_ref[:, border:border + hw] = xb * mm_ref[...]
            xp_ref[:, border:border + hw] = xb * mp_ref[...]

            for off, hlen in halves:
                for t in range(kh * kw):
                    ki, kj = divmod(t, kw)
                    st = border + (ki - 1) * w + (kj - 1) + off
                    src = (xm_ref if kj == 0
                           else (xp_ref if kj == kw - 1 else xe_ref))
                    p_ref[t * cin:(t + 1) * cin, :hlen] = src[:, st:st + hlen]

                acc = jax.lax.dot_general(
                    w_ref[...], p_ref[:, :hlen],
                    dimension_numbers=(((1,), (0,)), ((), ())),
                    preferred_element_type=jnp.float32)
                y = jnp.maximum(acc + b_ref[...], 0.0)
                y_ref[step * nb1 + b, :, off:off + hlen] = y.astype(jnp.bfloat16)

                s_ref[...] += jnp.sum(y, axis=1, keepdims=True)
                q_ref[...] += jnp.sum(y * y, axis=1, keepdims=True)

    @pl.when(step >= c1)
    def apply_phase():
        mean = s_ref[...] / m
        var = jnp.maximum(q_ref[...] / m - mean * mean, 0.0)
        inv = jax.lax.rsqrt(var + eps)
        scale = g_ref[...] * inv
        shift = be_ref[...] - mean * scale
        base = (step - c1) * nb2
        for b in range(nb2):
            yv = y_ref[base + b].astype(jnp.float32)
            o_ref[b] = yv * scale + shift


def kernel(x, weight, bias, gamma, beta):
    padding, eps = 1, 1e-5
    N, Cin, H, W = x.shape
    Cout, _, KH, KW = weight.shape
    assert KH == 3 and KW == 3 and padding == 1

    M = N * H * W
    K = KH * KW * Cin
    HW = H * W

    NB1 = 4 if N % 4 == 0 else 1
    NB2 = 8 if N % 8 == 0 else 1
    C1 = N // NB1
    C2 = N // NB2

    xflat = x.reshape(N, Cin, HW)

    w2 = jnp.transpose(weight, (0, 2, 3, 1)).reshape(Cout, K).astype(jnp.bfloat16)
    b2 = bias.reshape(Cout, 1).astype(jnp.float32)
    g2 = gamma.reshape(Cout, 1).astype(jnp.float32)
    be2 = beta.reshape(Cout, 1).astype(jnp.float32)

    lane = jnp.arange(HW, dtype=jnp.int32) % W
    maskm = (lane != W - 1).astype(jnp.bfloat16).reshape(1, HW)
    maskp = (lane != 0).astype(jnp.bfloat16).reshape(1, HW)

    body = functools.partial(_fused_kernel, nb1=NB1, nb2=NB2, c1=C1, cin=Cin,
                             kh=KH, kw=KW, h=H, w=W, m=float(M), eps=eps)
    out_flat = pl.pallas_call(
        body,
        out_shape=jax.ShapeDtypeStruct((N, Cout, HW), jnp.float32),
        grid_spec=pltpu.PrefetchScalarGridSpec(
            num_scalar_prefetch=0,
            grid=(C1 + C2,),
            in_specs=[
                pl.BlockSpec((NB1, Cin, HW),
                             lambda s: (jnp.minimum(s, C1 - 1), 0, 0)),
                pl.BlockSpec((Cout, K), lambda s: (0, 0)),
                pl.BlockSpec((Cout, 1), lambda s: (0, 0)),
                pl.BlockSpec((Cout, 1), lambda s: (0, 0)),
                pl.BlockSpec((Cout, 1), lambda s: (0, 0)),
                pl.BlockSpec((1, HW), lambda s: (0, 0)),
                pl.BlockSpec((1, HW), lambda s: (0, 0))],
            out_specs=pl.BlockSpec((NB2, Cout, HW),
                                   lambda s: (jnp.maximum(s - C1, 0), 0, 0)),
            scratch_shapes=[pltpu.VMEM((Cin, HW + 256 + 2 * (W + 1)), jnp.bfloat16),
                            pltpu.VMEM((Cin, HW + 256 + 2 * (W + 1)), jnp.bfloat16),
                            pltpu.VMEM((Cin, HW + 256 + 2 * (W + 1)), jnp.bfloat16),
                            pltpu.VMEM((K, HW), jnp.bfloat16),
                            pltpu.VMEM((N, Cout, HW), jnp.bfloat16),
                            pltpu.VMEM((Cout, 1), jnp.float32),
                            pltpu.VMEM((Cout, 1), jnp.float32)]),
        compiler_params=pltpu.CompilerParams(
            dimension_semantics=("arbitrary",),
            vmem_limit_bytes=60 * 1024 * 1024),
    )(xflat, w2, b2, g2, be2, maskm, maskp)

    return out_flat.reshape(N, Cout, H, W)


# speedup vs baseline: 1.0651x; 1.0651x over previous
import functools

import jax
import jax.numpy as jnp
from jax.experimental import pallas as pl
from jax.experimental.pallas import tpu as pltpu


def _fused_kernel(x_ref, w_ref, b_ref, g_ref, be_ref, mm_ref, mp_ref, o_ref,
                  xe_ref, xm_ref, xp_ref, p_ref, y_ref, s_ref, q_ref, *,
                  nb1, nb2, c1, cin, kh, kw, h, w, m, eps):
    step = pl.program_id(0)
    hw = h * w
    border = 128

    @pl.when(step < c1)
    def conv_phase():
        @pl.when(step == 0)
        def _init():
            xe_ref[...] = jnp.zeros_like(xe_ref)
            xm_ref[...] = jnp.zeros_like(xm_ref)
            xp_ref[...] = jnp.zeros_like(xp_ref)
            s_ref[...] = jnp.zeros_like(s_ref)
            q_ref[...] = jnp.zeros_like(q_ref)

        for b in range(nb1):
            xb = x_ref[b].astype(jnp.bfloat16)
            xe_ref[:, border:border + hw] = xb
            xm_ref[:, border:border + hw] = xb * mm_ref[...]
            xp_ref[:, border:border + hw] = xb * mp_ref[...]

            for t in range(kh * kw):
                ki, kj = divmod(t, kw)
                st = border + (ki - 1) * w + (kj - 1)
                src = xm_ref if kj == 0 else (xp_ref if kj == kw - 1 else xe_ref)
                p_ref[t * cin:(t + 1) * cin, :] = src[:, st:st + hw]

            acc = jax.lax.dot_general(
                w_ref[...], p_ref[...],
                dimension_numbers=(((1,), (0,)), ((), ())),
                preferred_element_type=jnp.float32)
            y = jnp.maximum(acc + b_ref[...], 0.0)
            y_ref[step * nb1 + b] = y.astype(jnp.bfloat16)

            s_ref[...] += jnp.sum(y, axis=1, keepdims=True)
            q_ref[...] += jnp.sum(y * y, axis=1, keepdims=True)

    @pl.when(step >= c1)
    def apply_phase():
        mean = s_ref[...] / m
        var = jnp.maximum(q_ref[...] / m - mean * mean, 0.0)
        inv = jax.lax.rsqrt(var + eps)
        scale = g_ref[...] * inv
        shift = be_ref[...] - mean * scale
        base = (step - c1) * nb2
        for b in range(nb2):
            yv = y_ref[base + b].astype(jnp.float32)
            o_ref[b] = yv * scale + shift


def kernel(x, weight, bias, gamma, beta):
    padding, eps = 1, 1e-5
    N, Cin, H, W = x.shape
    Cout, _, KH, KW = weight.shape
    assert KH == 3 and KW == 3 and padding == 1

    M = N * H * W
    K = KH * KW * Cin
    HW = H * W

    NB1 = 4 if N % 4 == 0 else 1
    NB2 = 8 if N % 8 == 0 else 1
    C1 = N // NB1
    C2 = N // NB2

    xflat = x.reshape(N, Cin, HW)

    w2 = jnp.transpose(weight, (0, 2, 3, 1)).reshape(Cout, K).astype(jnp.bfloat16)
    b2 = bias.reshape(Cout, 1).astype(jnp.float32)
    g2 = gamma.reshape(Cout, 1).astype(jnp.float32)
    be2 = beta.reshape(Cout, 1).astype(jnp.float32)

    lane = jnp.arange(HW, dtype=jnp.int32) % W
    maskm = (lane != W - 1).astype(jnp.bfloat16).reshape(1, HW)
    maskp = (lane != 0).astype(jnp.bfloat16).reshape(1, HW)

    body = functools.partial(_fused_kernel, nb1=NB1, nb2=NB2, c1=C1, cin=Cin,
                             kh=KH, kw=KW, h=H, w=W, m=float(M), eps=eps)
    out_flat = pl.pallas_call(
        body,
        out_shape=jax.ShapeDtypeStruct((N, Cout, HW), jnp.float32),
        grid_spec=pltpu.PrefetchScalarGridSpec(
            num_scalar_prefetch=0,
            grid=(C1 + C2,),
            in_specs=[
                pl.BlockSpec((NB1, Cin, HW),
                             lambda s: (jnp.minimum(s, C1 - 1), 0, 0)),
                pl.BlockSpec((Cout, K), lambda s: (0, 0)),
                pl.BlockSpec((Cout, 1), lambda s: (0, 0)),
                pl.BlockSpec((Cout, 1), lambda s: (0, 0)),
                pl.BlockSpec((Cout, 1), lambda s: (0, 0)),
                pl.BlockSpec((1, HW), lambda s: (0, 0)),
                pl.BlockSpec((1, HW), lambda s: (0, 0))],
            out_specs=pl.BlockSpec((NB2, Cout, HW),
                                   lambda s: (jnp.maximum(s - C1, 0), 0, 0)),
            scratch_shapes=[pltpu.VMEM((Cin, HW + 256 + 2 * (W + 1)), jnp.bfloat16),
                            pltpu.VMEM((Cin, HW + 256 + 2 * (W + 1)), jnp.bfloat16),
                            pltpu.VMEM((Cin, HW + 256 + 2 * (W + 1)), jnp.bfloat16),
                            pltpu.VMEM((K, HW), jnp.bfloat16),
                            pltpu.VMEM((N, Cout, HW), jnp.bfloat16),
                            pltpu.VMEM((Cout, 1), jnp.float32),
                            pltpu.VMEM((Cout, 1), jnp.float32)]),
        compiler_params=pltpu.CompilerParams(
            dimension_semantics=("arbitrary",),
            vmem_limit_bytes=60 * 1024 * 1024),
    )(xflat, w2, b2, g2, be2, maskm, maskp)

    return out_flat.reshape(N, Cout, H, W)
